# baseline (speedup 1.0000x reference)
import functools

import jax
import jax.numpy as jnp
from jax import lax
from jax.experimental import pallas as pl
from jax.experimental.pallas import tpu as pltpu
from jax.experimental.pallas import tpu_sc as plsc

N = 10000
E = 320000
IN_CH = 128
HID = 256
OUT_CH = 128

NC = 2
NS = 16
L = 16
CH = HID // NC
K = 80
NCHUNK = E // K
SLAB = 624
NF = NCHUNK // NS
NFD = NCHUNK // (NC * NS)
NBUF = 4
NDI = 8
RBLK = 200

_mesh = functools.partial(
    plsc.VectorSubcoreMesh, core_axis_name="c", subcore_axis_name="s",
    num_cores=NC, num_subcores=NS)


def _deg_body(dstp_hbm, deg_hbm, dstage, ones_v, zb, dacc):
    c = lax.axis_index("c")
    s = lax.axis_index("s")

    zrow = jnp.zeros((L,), jnp.float32)
    onehot = jnp.where(lax.iota(jnp.int32, L) == 0, 1.0, 0.0)
    for r in range(L):
        zb[r] = zrow
    for r in range(K):
        ones_v[r] = onehot

    base = s * SLAB
    for j in range(SLAB // L):
        pltpu.sync_copy(zb, dacc.at[pl.ds(base + j * L, L)])

    @pl.when(s == NS - 1)
    def _():
        pltpu.sync_copy(zb, dacc.at[pl.ds(NS * SLAB, L)])

    r0 = (c * NS + s) * NFD
    pltpu.sync_copy(dstp_hbm.at[pl.ds(r0, NFD)], dstage)

    plsc.subcore_barrier()

    def body(j):
        pltpu.sync_copy(ones_v, dacc.at[dstage.at[j, 0]], add=True)

    pl.loop(0, NFD)(body)

    plsc.subcore_barrier()
    pltpu.sync_copy(dacc.at[pl.ds(base, SLAB)],
                    deg_hbm.at[c, pl.ds(base, SLAB)])

    @pl.when(s == NS - 1)
    def _():
        pltpu.sync_copy(dacc.at[pl.ds(NS * SLAB, L)],
                        deg_hbm.at[c, pl.ds(NS * SLAB, L)])


def _deg(dstp):
    return pl.kernel(
        _deg_body,
        out_type=jax.ShapeDtypeStruct((NC, N, L), jnp.float32),
        mesh=_mesh(),
        scratch_types=[
            pltpu.VMEM((NFD, 1, K), jnp.int32),
            pltpu.VMEM((K, L), jnp.float32),
            pltpu.VMEM((L, L), jnp.float32),
            pltpu.VMEM_SHARED((N, L), jnp.float32),
        ],
        name="gcn_deg_sc",
    )(dstp)


def _msg_body(g_hbm, srcp_hbm, dstp_hbm, out_hbm,
              isr, idr, gbuf, sem_i, sem_g, sem_s, macc):
    c = lax.axis_index("c")
    s = lax.axis_index("s")
    sbase = c * NCHUNK + s * NF
    dbase = s * NF
    base = s * SLAB

    zrow = jnp.zeros((L,), jnp.float32)
    for r in range(K):
        for qq in range(CH // L):
            gbuf[0, r, pl.ds(qq * L, L)] = zrow
    for j in range(SLAB // K):
        pltpu.sync_copy(gbuf.at[0], macc.at[pl.ds(base + j * K, K)])
    rem = SLAB % K
    if rem:
        pltpu.sync_copy(gbuf.at[0, pl.ds(0, rem)],
                        macc.at[pl.ds(base + SLAB - rem, rem)])

    @pl.when(s == NS - 1)
    def _():
        pltpu.sync_copy(gbuf.at[0, pl.ds(0, L)],
                        macc.at[pl.ds(NS * SLAB, L)])

    for i in range(NDI - 2):
        pltpu.async_copy(srcp_hbm.at[pl.ds(sbase + i, 1)],
                         isr.at[pl.ds(i, 1)], sem_i)
        pltpu.async_copy(dstp_hbm.at[pl.ds(dbase + i, 1)],
                         idr.at[pl.ds(i, 1)], sem_i)

    plsc.subcore_barrier()

    for i in range(2):
        pltpu.make_async_copy(srcp_hbm.at[pl.ds(0, 1)],
                              isr.at[pl.ds(i, 1)], sem_i).wait()
        pltpu.make_async_copy(dstp_hbm.at[pl.ds(0, 1)],
                              idr.at[pl.ds(i, 1)], sem_i).wait()
        pltpu.async_copy(g_hbm.at[isr.at[i, 0]], gbuf.at[i], sem_g)

    def step(j, b8, in_loop, last=False):
        b4 = b8 % NBUF
        pltpu.make_async_copy(g_hbm.at[pl.ds(0, K)],
                              gbuf.at[b4], sem_g).wait()
        if last:
            pltpu.sync_copy(gbuf.at[b4], macc.at[idr.at[b8, 0]], add=True)
        else:
            pltpu.async_copy(gbuf.at[b4], macc.at[idr.at[b8, 0]], sem_s,
                             add=True)

        def wait_sc():
            pltpu.make_async_copy(gbuf.at[(b4 + 3) % NBUF],
                                  macc.at[idr.at[(b8 + 7) % NDI, 0]],
                                  sem_s).wait()

        if in_loop:
            pl.when(j >= 1)(wait_sc)
        elif isinstance(j, int) and j >= 1:
            wait_sc()

        def issue_g():
            pltpu.make_async_copy(srcp_hbm.at[pl.ds(0, 1)],
                                  isr.at[pl.ds((b8 + 2) % NDI, 1)],
                                  sem_i).wait()
            pltpu.make_async_copy(dstp_hbm.at[pl.ds(0, 1)],
                                  idr.at[pl.ds((b8 + 2) % NDI, 1)],
                                  sem_i).wait()
            pltpu.async_copy(g_hbm.at[isr.at[(b8 + 2) % NDI, 0]],
                             gbuf.at[(b4 + 2) % NBUF], sem_g)

        if in_loop:
            pl.when(j + 2 < NF)(issue_g)
        elif isinstance(j, int) and j + 2 < NF:
            issue_g()

        def issue_i():
            pltpu.async_copy(srcp_hbm.at[pl.ds(sbase + j + 6, 1)],
                             isr.at[pl.ds((b8 + 6) % NDI, 1)], sem_i)
            pltpu.async_copy(dstp_hbm.at[pl.ds(dbase + j + 6, 1)],
                             idr.at[pl.ds((b8 + 6) % NDI, 1)], sem_i)

        if in_loop:
            pl.when(j + 6 < NF)(issue_i)
        elif isinstance(j, int) and j + 6 < NF:
            issue_i()

    def outer(qq):
        for b8 in range(NDI):
            step(qq * NDI + b8, b8, True)

    nloop = (NF // NDI) * NDI
    pl.loop(0, nloop // NDI)(outer)
    for j in range(nloop, NF):
        step(j, j % NDI, False, last=(j == NF - 1))

    plsc.subcore_barrier()
    pltpu.sync_copy(macc.at[pl.ds(base, SLAB)],
                    out_hbm.at[c, pl.ds(base, SLAB)])

    @pl.when(s == NS - 1)
    def _():
        pltpu.sync_copy(macc.at[pl.ds(NS * SLAB, L)],
                        out_hbm.at[c, pl.ds(NS * SLAB, L)])


def _msg(g2, srcp, dstp):
    return pl.kernel(
        _msg_body,
        out_type=jax.ShapeDtypeStruct((NC, N, CH), jnp.float32),
        mesh=_mesh(),
        scratch_types=[
            pltpu.VMEM((NDI, 1, K), jnp.int32),
            pltpu.VMEM((NDI, 1, K), jnp.int32),
            pltpu.VMEM((NBUF, K, CH), jnp.float32),
            pltpu.SemaphoreType.DMA,
            pltpu.SemaphoreType.DMA,
            pltpu.SemaphoreType.DMA,
            pltpu.VMEM_SHARED((N, CH), jnp.float32),
        ],
        name="gcn_msg_sc",
    )(g2, srcp, dstp)


def _elu(v):
    return jnp.where(v > 0, v, jnp.exp(v) - 1.0)


def _dinv_of(deg_blk):
    return lax.rsqrt(deg_blk[0, :, 0:1] + deg_blk[1, :, 0:1] + 1.0)


def _split_g(gfull):
    return jnp.stack([gfull[:, :CH], gfull[:, CH:]], axis=0)


def _join_msg(msg_blk):
    return jnp.concatenate([msg_blk[0], msg_blk[1]], axis=-1)


def _tc1_body(x_ref, deg_ref, win_ref, bin_ref, w1_ref, g_ref):
    h = jnp.dot(x_ref[...], win_ref[...], preferred_element_type=jnp.float32)
    h = _elu(h + bin_ref[...])
    hp = jnp.dot(h, w1_ref[...], preferred_element_type=jnp.float32)
    g_ref[...] = _split_g(_dinv_of(deg_ref[...]) * hp)


def _tc_mid_body(msg_ref, g_ref, deg_ref, b_ref, w_ref, out_ref):
    dinv = _dinv_of(deg_ref[...])
    pre = dinv * (_join_msg(msg_ref[...]) + _join_msg(g_ref[...])) + b_ref[...]
    h = _elu(pre)
    hp = jnp.dot(h, w_ref[...], preferred_element_type=jnp.float32)
    out_ref[...] = _split_g(dinv * hp)


def _tc_out_body(msg_ref, g_ref, deg_ref, b_ref, wout_ref, bout_ref, out_ref):
    dinv = _dinv_of(deg_ref[...])
    pre = dinv * (_join_msg(msg_ref[...]) + _join_msg(g_ref[...])) + b_ref[...]
    h = _elu(pre)
    out_ref[...] = jnp.dot(h, wout_ref[...],
                           preferred_element_type=jnp.float32) + bout_ref[...]


_NBLK = N // RBLK


def _deg_spec():
    return pl.BlockSpec((NC, RBLK, L), lambda i: (0, i, 0))


def _half_spec():
    return pl.BlockSpec((NC, RBLK, CH), lambda i: (0, i, 0))


def _row_spec(ch):
    return pl.BlockSpec((RBLK, ch), lambda i: (i, 0))


def _full_spec(r, c_):
    return pl.BlockSpec((r, c_), lambda i: (0, 0))


def _tc1(x, deg, W_in, b_in, W1):
    return pl.pallas_call(
        _tc1_body,
        grid=(_NBLK,),
        in_specs=[_row_spec(IN_CH), _deg_spec(), _full_spec(IN_CH, HID),
                  _full_spec(1, HID), _full_spec(HID, HID)],
        out_specs=_half_spec(),
        out_shape=jax.ShapeDtypeStruct((NC, N, CH), jnp.float32),
    )(x, deg, W_in, b_in, W1)


def _tc_mid(msg, g, deg, b, W):
    return pl.pallas_call(
        _tc_mid_body,
        grid=(_NBLK,),
        in_specs=[_half_spec(), _half_spec(), _deg_spec(),
                  _full_spec(1, HID), _full_spec(HID, HID)],
        out_specs=_half_spec(),
        out_shape=jax.ShapeDtypeStruct((NC, N, CH), jnp.float32),
    )(msg, g, deg, b, W)


def _tc_out(msg, g, deg, b, W_out, b_out):
    return pl.pallas_call(
        _tc_out_body,
        grid=(_NBLK,),
        in_specs=[_half_spec(), _half_spec(), _deg_spec(),
                  _full_spec(1, HID), _full_spec(HID, OUT_CH),
                  _full_spec(1, OUT_CH)],
        out_specs=_row_spec(OUT_CH),
        out_shape=jax.ShapeDtypeStruct((N, OUT_CH), jnp.float32),
    )(msg, g, deg, b, W_out, b_out)


def kernel(x, adj, W_in, b_in, W1, b1, W2, b2, W_out, b_out):
    src = adj[0].astype(jnp.int32)
    dst = adj[1].astype(jnp.int32)
    src2d = src.reshape(NCHUNK, K)
    offs = (jnp.arange(NC, dtype=jnp.int32) * N)[:, None, None]
    srcp = (src2d[None] + offs).reshape(NC * NCHUNK, 1, K)
    dstp = dst.reshape(NCHUNK, 1, K)

    deg = _deg(dstp)
    g1 = _tc1(x, deg, W_in, b_in.reshape(1, HID), W1)
    m1 = _msg(g1.reshape(NC * N, CH), srcp, dstp)
    g2 = _tc_mid(m1, g1, deg, b1.reshape(1, HID), W2)
    m2 = _msg(g2.reshape(NC * N, CH), srcp, dstp)
    out = _tc_out(m2, g2, deg, b2.reshape(1, HID), W_out,
                  b_out.reshape(1, OUT_CH))
    return out


# speedup vs baseline: 1.0901x; 1.0901x over previous
import functools

import jax
import jax.numpy as jnp
from jax import lax
from jax.experimental import pallas as pl
from jax.experimental.pallas import tpu as pltpu
from jax.experimental.pallas import tpu_sc as plsc

N = 10000
E = 320000
IN_CH = 128
HID = 256
OUT_CH = 128

NC = 2
NS = 16
L = 16
CH = HID // NC
K = 80
NCHUNK = E // K
SLAB = 624
NF = NCHUNK // NS
NFD = NCHUNK // (NC * NS)
NBUF = 4
NDI = 8
RBLK = 400

_mesh = functools.partial(
    plsc.VectorSubcoreMesh, core_axis_name="c", subcore_axis_name="s",
    num_cores=NC, num_subcores=NS)


def _deg_body(dstp_hbm, deg_hbm, dstage, ones_v, zb, dsem, dacc):
    c = lax.axis_index("c")
    s = lax.axis_index("s")

    zrow = jnp.zeros((L,), jnp.float32)
    onehot = jnp.where(lax.iota(jnp.int32, L) == 0, 1.0, 0.0)
    for r in range(L):
        zb[r] = zrow
    for r in range(K):
        ones_v[r] = onehot

    base = s * SLAB
    for j in range(SLAB // L):
        pltpu.sync_copy(zb, dacc.at[pl.ds(base + j * L, L)])

    @pl.when(s == NS - 1)
    def _():
        pltpu.sync_copy(zb, dacc.at[pl.ds(NS * SLAB, L)])

    r0 = (c * NS + s) * NFD
    pltpu.sync_copy(dstp_hbm.at[pl.ds(r0, NFD)], dstage)

    plsc.subcore_barrier()

    def body(j):
        pltpu.async_copy(ones_v, dacc.at[dstage.at[j, 0]], dsem, add=True)

        @pl.when(j >= 4)
        def _():
            pltpu.make_async_copy(ones_v, dacc.at[dstage.at[0, 0]],
                                  dsem).wait()

    pl.loop(0, NFD - 1)(body)
    for _ in range(4):
        pltpu.make_async_copy(ones_v, dacc.at[dstage.at[0, 0]], dsem).wait()
    pltpu.sync_copy(ones_v, dacc.at[dstage.at[NFD - 1, 0]], add=True)

    plsc.subcore_barrier()
    pltpu.sync_copy(dacc.at[pl.ds(base, SLAB)],
                    deg_hbm.at[c, pl.ds(base, SLAB)])

    @pl.when(s == NS - 1)
    def _():
        pltpu.sync_copy(dacc.at[pl.ds(NS * SLAB, L)],
                        deg_hbm.at[c, pl.ds(NS * SLAB, L)])


def _deg(dstp):
    return pl.kernel(
        _deg_body,
        out_type=jax.ShapeDtypeStruct((NC, N, L), jnp.float32),
        mesh=_mesh(),
        scratch_types=[
            pltpu.VMEM((NFD, 1, K), jnp.int32),
            pltpu.VMEM((K, L), jnp.float32),
            pltpu.VMEM((L, L), jnp.float32),
            pltpu.SemaphoreType.DMA,
            pltpu.VMEM_SHARED((N, L), jnp.float32),
        ],
        name="gcn_deg_sc",
    )(dstp)


def _msg_body(g_hbm, srcp_hbm, dstp_hbm, out_hbm,
              isr, idr, gbuf, sem_i, sem_g, sem_s, macc):
    c = lax.axis_index("c")
    s = lax.axis_index("s")
    sbase = c * NCHUNK + s * NF
    dbase = s * NF
    base = s * SLAB

    zrow = jnp.zeros((L,), jnp.float32)
    for r in range(K):
        for qq in range(CH // L):
            gbuf[0, r, pl.ds(qq * L, L)] = zrow
    for j in range(SLAB // K):
        pltpu.sync_copy(gbuf.at[0], macc.at[pl.ds(base + j * K, K)])
    rem = SLAB % K
    if rem:
        pltpu.sync_copy(gbuf.at[0, pl.ds(0, rem)],
                        macc.at[pl.ds(base + SLAB - rem, rem)])

    @pl.when(s == NS - 1)
    def _():
        pltpu.sync_copy(gbuf.at[0, pl.ds(0, L)],
                        macc.at[pl.ds(NS * SLAB, L)])

    for i in range(NDI - 2):
        pltpu.async_copy(srcp_hbm.at[pl.ds(sbase + i, 1)],
                         isr.at[pl.ds(i, 1)], sem_i)
        pltpu.async_copy(dstp_hbm.at[pl.ds(dbase + i, 1)],
                         idr.at[pl.ds(i, 1)], sem_i)

    plsc.subcore_barrier()

    for i in range(2):
        pltpu.make_async_copy(srcp_hbm.at[pl.ds(0, 1)],
                              isr.at[pl.ds(i, 1)], sem_i).wait()
        pltpu.make_async_copy(dstp_hbm.at[pl.ds(0, 1)],
                              idr.at[pl.ds(i, 1)], sem_i).wait()
        pltpu.async_copy(g_hbm.at[isr.at[i, 0]], gbuf.at[i], sem_g)

    def step(j, b8, in_loop, last=False):
        b4 = b8 % NBUF
        pltpu.make_async_copy(g_hbm.at[pl.ds(0, K)],
                              gbuf.at[b4], sem_g).wait()
        if last:
            pltpu.sync_copy(gbuf.at[b4], macc.at[idr.at[b8, 0]], add=True)
        else:
            pltpu.async_copy(gbuf.at[b4], macc.at[idr.at[b8, 0]], sem_s,
                             add=True)

        def wait_sc():
            pltpu.make_async_copy(gbuf.at[(b4 + 3) % NBUF],
                                  macc.at[idr.at[(b8 + 7) % NDI, 0]],
                                  sem_s).wait()

        if in_loop:
            pl.when(j >= 1)(wait_sc)
        elif isinstance(j, int) and j >= 1:
            wait_sc()

        def issue_g():
            pltpu.make_async_copy(srcp_hbm.at[pl.ds(0, 1)],
                                  isr.at[pl.ds((b8 + 2) % NDI, 1)],
                                  sem_i).wait()
            pltpu.make_async_copy(dstp_hbm.at[pl.ds(0, 1)],
                                  idr.at[pl.ds((b8 + 2) % NDI, 1)],
                                  sem_i).wait()
            pltpu.async_copy(g_hbm.at[isr.at[(b8 + 2) % NDI, 0]],
                             gbuf.at[(b4 + 2) % NBUF], sem_g)

        if in_loop:
            pl.when(j + 2 < NF)(issue_g)
        elif isinstance(j, int) and j + 2 < NF:
            issue_g()

        def issue_i():
            pltpu.async_copy(srcp_hbm.at[pl.ds(sbase + j + 6, 1)],
                             isr.at[pl.ds((b8 + 6) % NDI, 1)], sem_i)
            pltpu.async_copy(dstp_hbm.at[pl.ds(dbase + j + 6, 1)],
                             idr.at[pl.ds((b8 + 6) % NDI, 1)], sem_i)

        if in_loop:
            pl.when(j + 6 < NF)(issue_i)
        elif isinstance(j, int) and j + 6 < NF:
            issue_i()

    def outer(qq):
        for b8 in range(NDI):
            step(qq * NDI + b8, b8, True)

    nloop = (NF // NDI) * NDI
    pl.loop(0, nloop // NDI)(outer)
    for j in range(nloop, NF):
        step(j, j % NDI, False, last=(j == NF - 1))

    plsc.subcore_barrier()
    pltpu.sync_copy(macc.at[pl.ds(base, SLAB)],
                    out_hbm.at[c, pl.ds(base, SLAB)])

    @pl.when(s == NS - 1)
    def _():
        pltpu.sync_copy(macc.at[pl.ds(NS * SLAB, L)],
                        out_hbm.at[c, pl.ds(NS * SLAB, L)])


def _msg(g2, srcp, dstp):
    return pl.kernel(
        _msg_body,
        out_type=jax.ShapeDtypeStruct((NC, N, CH), jnp.float32),
        mesh=_mesh(),
        scratch_types=[
            pltpu.VMEM((NDI, 1, K), jnp.int32),
            pltpu.VMEM((NDI, 1, K), jnp.int32),
            pltpu.VMEM((NBUF, K, CH), jnp.float32),
            pltpu.SemaphoreType.DMA,
            pltpu.SemaphoreType.DMA,
            pltpu.SemaphoreType.DMA,
            pltpu.VMEM_SHARED((N, CH), jnp.float32),
        ],
        name="gcn_msg_sc",
    )(g2, srcp, dstp)


def _elu(v):
    return jnp.where(v > 0, v, jnp.exp(v) - 1.0)


def _dinv_of(deg_blk):
    return lax.rsqrt(deg_blk[0, :, 0:1] + deg_blk[1, :, 0:1] + 1.0)


def _split_g(gfull):
    return jnp.stack([gfull[:, :CH], gfull[:, CH:]], axis=0)


def _join_msg(msg_blk):
    return jnp.concatenate([msg_blk[0], msg_blk[1]], axis=-1)


def _tc1_body(x_ref, deg_ref, win_ref, bin_ref, w1_ref, g_ref):
    h = jnp.dot(x_ref[...], win_ref[...], preferred_element_type=jnp.float32)
    h = _elu(h + bin_ref[...])
    hp = jnp.dot(h, w1_ref[...], preferred_element_type=jnp.float32)
    g_ref[...] = _split_g(_dinv_of(deg_ref[...]) * hp)


def _tc_mid_body(msg_ref, g_ref, deg_ref, b_ref, w_ref, out_ref):
    dinv = _dinv_of(deg_ref[...])
    pre = dinv * (_join_msg(msg_ref[...]) + _join_msg(g_ref[...])) + b_ref[...]
    h = _elu(pre)
    hp = jnp.dot(h, w_ref[...], preferred_element_type=jnp.float32)
    out_ref[...] = _split_g(dinv * hp)


def _tc_out_body(msg_ref, g_ref, deg_ref, b_ref, wout_ref, bout_ref, out_ref):
    dinv = _dinv_of(deg_ref[...])
    pre = dinv * (_join_msg(msg_ref[...]) + _join_msg(g_ref[...])) + b_ref[...]
    h = _elu(pre)
    out_ref[...] = jnp.dot(h, wout_ref[...],
                           preferred_element_type=jnp.float32) + bout_ref[...]


_NBLK = N // RBLK


def _deg_spec():
    return pl.BlockSpec((NC, RBLK, L), lambda i: (0, i, 0))


def _half_spec():
    return pl.BlockSpec((NC, RBLK, CH), lambda i: (0, i, 0))


def _row_spec(ch):
    return pl.BlockSpec((RBLK, ch), lambda i: (i, 0))


def _full_spec(r, c_):
    return pl.BlockSpec((r, c_), lambda i: (0, 0))


def _tc1(x, deg, W_in, b_in, W1):
    return pl.pallas_call(
        _tc1_body,
        grid=(_NBLK,),
        in_specs=[_row_spec(IN_CH), _deg_spec(), _full_spec(IN_CH, HID),
                  _full_spec(1, HID), _full_spec(HID, HID)],
        out_specs=_half_spec(),
        out_shape=jax.ShapeDtypeStruct((NC, N, CH), jnp.float32),
    )(x, deg, W_in, b_in, W1)


def _tc_mid(msg, g, deg, b, W):
    return pl.pallas_call(
        _tc_mid_body,
        grid=(_NBLK,),
        in_specs=[_half_spec(), _half_spec(), _deg_spec(),
                  _full_spec(1, HID), _full_spec(HID, HID)],
        out_specs=_half_spec(),
        out_shape=jax.ShapeDtypeStruct((NC, N, CH), jnp.float32),
    )(msg, g, deg, b, W)


def _tc_out(msg, g, deg, b, W_out, b_out):
    return pl.pallas_call(
        _tc_out_body,
        grid=(_NBLK,),
        in_specs=[_half_spec(), _half_spec(), _deg_spec(),
                  _full_spec(1, HID), _full_spec(HID, OUT_CH),
                  _full_spec(1, OUT_CH)],
        out_specs=_row_spec(OUT_CH),
        out_shape=jax.ShapeDtypeStruct((N, OUT_CH), jnp.float32),
    )(msg, g, deg, b, W_out, b_out)


def kernel(x, adj, W_in, b_in, W1, b1, W2, b2, W_out, b_out):
    src = adj[0].astype(jnp.int32)
    dst = adj[1].astype(jnp.int32)
    src2d = src.reshape(NCHUNK, K)
    offs = (jnp.arange(NC, dtype=jnp.int32) * N)[:, None, None]
    srcp = (src2d[None] + offs).reshape(NC * NCHUNK, 1, K)
    dstp = dst.reshape(NCHUNK, 1, K)

    deg = _deg(dstp)
    g1 = _tc1(x, deg, W_in, b_in.reshape(1, HID), W1)
    m1 = _msg(g1.reshape(NC * N, CH), srcp, dstp)
    g2 = _tc_mid(m1, g1, deg, b1.reshape(1, HID), W2)
    m2 = _msg(g2.reshape(NC * N, CH), srcp, dstp)
    out = _tc_out(m2, g2, deg, b2.reshape(1, HID), W_out,
                  b_out.reshape(1, OUT_CH))
    return out
